# baseline (speedup 1.0000x reference)
import functools

import jax
import jax.numpy as jnp
from jax import lax
from jax.experimental import pallas as pl
from jax.experimental.pallas import tpu as pltpu
from jax.experimental.pallas import tpu_sc as plsc

_A = 26
_L = 20
_D = 64
_P = _A * _L
_TILE = 128
_NC = 2
_NS = 16
_NW = _NC * _NS
_EPS = 1e-5


def _kernel_body(xr_hbm, tok_hbm, av_hbm, pos_hbm, gam_hbm, bet_hbm, out_hbm,
                 idx_v, av_v, pos_v, avpos, gam_v, bet_v, buf, bufo,
                 sem_in0, sem_in1, sem_out0, sem_out1):
  tiles_per_w = xr_hbm.shape[1] // _TILE
  wid = lax.axis_index("s") * _NC + lax.axis_index("c")

  pltpu.sync_copy(xr_hbm.at[wid], idx_v)
  pltpu.sync_copy(av_hbm, av_v)
  pltpu.sync_copy(pos_hbm, pos_v)
  pltpu.sync_copy(gam_hbm, gam_v)
  pltpu.sync_copy(bet_hbm, bet_v)

  def build_avpos(r, _):
    a = r // _L
    l = r - a * _L
    for dd in range(_D // 16):
      sl = pl.ds(dd * 16, 16)
      avpos[r, sl] = av_v[a, sl] + pos_v[l, sl]
    return 0
  lax.fori_loop(0, _P, build_avpos, 0)

  g = [gam_v[pl.ds(dd * 16, 16)] for dd in range(_D // 16)]
  b = [bet_v[pl.ds(dd * 16, 16)] for dd in range(_D // 16)]

  sems_in = (sem_in0, sem_in1)
  sems_out = (sem_out0, sem_out1)

  def start_gather(t, s):
    pltpu.async_copy(tok_hbm.at[idx_v.at[pl.ds(t * _TILE, _TILE)]],
                     buf.at[s], sems_in[s])

  def wait_gather(t, s):
    pltpu.make_async_copy(tok_hbm.at[idx_v.at[pl.ds(t * _TILE, _TILE)]],
                          buf.at[s], sems_in[s]).wait()

  def start_out(t, s):
    row0 = (wid * tiles_per_w + t) * (_TILE // 2)
    pltpu.async_copy(bufo.at[s], out_hbm.at[pl.ds(row0, _TILE // 2)],
                     sems_out[s])

  def wait_out(t, s):
    row0 = (wid * tiles_per_w + t) * (_TILE // 2)
    pltpu.make_async_copy(bufo.at[s], out_hbm.at[pl.ds(row0, _TILE // 2)],
                          sems_out[s]).wait()

  def compute_tile(t, s):
    tok0 = (wid * tiles_per_w + t) * _TILE
    base_r = lax.rem(tok0, _P)

    @plsc.parallel_loop(0, _TILE, 1, unroll=8)
    def tok(j):
      r0 = base_r + j
      r = jnp.where(r0 >= _P, r0 - _P, r0)
      x = []
      for dd in range(_D // 16):
        sl = pl.ds(dd * 16, 16)
        x.append(buf[s, j, sl] + avpos[r, sl])
      sm = (x[0] + x[1]) + (x[2] + x[3])
      sq = (x[0] * x[0] + x[1] * x[1]) + (x[2] * x[2] + x[3] * x[3])
      ssum = jnp.sum(sm)
      qsum = jnp.sum(sq)
      mean = ssum * (1.0 / _D)
      var = qsum * (1.0 / _D) - mean * mean + _EPS
      iv = lax.bitcast_convert_type(var, jnp.int32)
      iv = jnp.int32(0x5F3759DF) - lax.shift_right_logical(iv, 1)
      y = lax.bitcast_convert_type(iv, jnp.float32)
      h = var * 0.5
      y = y * (1.5 - h * y * y)
      y = y * (1.5 - h * y * y)
      y = y * (1.5 - h * y * y)
      c0 = mean * y
      jr = j >> 1
      jc = (j & 1) * _D
      for dd in range(_D // 16):
        bufo[s, jr, pl.ds(jc + dd * 16, 16)] = (
            (x[dd] * y - c0) * g[dd] + b[dd])

  start_gather(0, 0)

  def outer(tt, _):
    for s in range(2):
      t = tt * 2 + s
      wait_gather(t, s)
      compute_tile(t, s)
      start_out(t, s)
      nxt = 1 - s
      if s == 0:
        @pl.when(tt >= 1)
        def _():
          wait_out(t - 1, nxt)
        start_gather(t + 1, nxt)
      else:
        @pl.when(tt < tiles_per_w // 2 - 1)
        def _():
          wait_out(t - 1, nxt)
          start_gather(t + 1, nxt)
    return 0

  lax.fori_loop(0, tiles_per_w // 2, outer, 0)
  wait_out(tiles_per_w - 2, 0)
  wait_out(tiles_per_w - 1, 1)


@jax.jit
def kernel(X_scan, token_table, av_table, pos_table, ln_gamma, ln_beta):
  rows, seq = X_scan.shape
  n = rows * seq
  n_tiles = n // _TILE
  xr = X_scan.reshape(_NW, n // _NW).astype(jnp.int32)

  mesh = plsc.VectorSubcoreMesh(
      core_axis_name="c", subcore_axis_name="s",
      num_cores=_NC, num_subcores=_NS)
  tiles_per_w = n_tiles // _NW

  run = pl.kernel(
      _kernel_body,
      out_type=jax.ShapeDtypeStruct((n * _D // 128, 128), jnp.float32),
      mesh=mesh,
      scratch_types=[
          pltpu.VMEM((n // _NW,), jnp.int32),
          pltpu.VMEM((_A, _D), jnp.float32),
          pltpu.VMEM((_L, _D), jnp.float32),
          pltpu.VMEM((_P, _D), jnp.float32),
          pltpu.VMEM((_D,), jnp.float32),
          pltpu.VMEM((_D,), jnp.float32),
          pltpu.VMEM((2, _TILE, _D), jnp.float32),
          pltpu.VMEM((2, _TILE // 2, 2 * _D), jnp.float32),
          pltpu.SemaphoreType.DMA,
          pltpu.SemaphoreType.DMA,
          pltpu.SemaphoreType.DMA,
          pltpu.SemaphoreType.DMA,
      ],
      compiler_params=pltpu.CompilerParams(
          needs_layout_passes=False, use_tc_tiling_on_sc=False),
  )
  out = run(xr, token_table, av_table, pos_table, ln_gamma, ln_beta)
  return out.reshape(rows, seq, _D)


# speedup vs baseline: 1.0597x; 1.0597x over previous
import jax
import jax.numpy as jnp
from jax import lax
from jax.experimental import pallas as pl
from jax.experimental.pallas import tpu as pltpu
from jax.experimental.pallas import tpu_sc as plsc

_A = 26
_L = 20
_D = 64
_P = _A * _L
_TILE = 160
_ROWS_T = _TILE // _L
_NC = 2
_NS = 16
_NW = _NC * _NS
_EPS = 1e-5


def _kernel_body(xr_hbm, tok_hbm, av_hbm, pos_hbm, gam_hbm, bet_hbm, out_hbm,
                 idx_v, av_v, pos_v, avpos, gam_v, bet_v, buf, bufo,
                 sem_in0, sem_in1, sem_out0, sem_out1):
  n_per_w = xr_hbm.shape[0] // _NW
  tiles_per_w = n_per_w // _TILE
  rows_per_w = n_per_w // _L
  wid = lax.axis_index("s") * _NC + lax.axis_index("c")

  pltpu.sync_copy(xr_hbm.at[pl.ds(wid * n_per_w, n_per_w)], idx_v)
  pltpu.sync_copy(av_hbm, av_v)
  pltpu.sync_copy(pos_hbm, pos_v)
  pltpu.sync_copy(gam_hbm, gam_v)
  pltpu.sync_copy(bet_hbm, bet_v)

  def build_avpos(r, _):
    a = r // _L
    l = r - a * _L
    for dd in range(_D // 16):
      sl = pl.ds(dd * 16, 16)
      avpos[r, sl] = av_v[a, sl] + pos_v[l, sl]
    return 0
  lax.fori_loop(0, _P, build_avpos, 0)

  g = [gam_v[pl.ds(dd * 16, 16)] for dd in range(_D // 16)]
  b = [bet_v[pl.ds(dd * 16, 16)] for dd in range(_D // 16)]

  sems_in = (sem_in0, sem_in1)
  sems_out = (sem_out0, sem_out1)
  _H = _TILE // 2

  def start_gather(t, s):
    for h in range(2):
      pltpu.async_copy(
          tok_hbm.at[idx_v.at[pl.ds(t * _TILE + h * _H, _H)]],
          buf.at[s, pl.ds(h * _H, _H)], sems_in[s])

  def wait_gather(t, s):
    for h in range(2):
      pltpu.make_async_copy(
          tok_hbm.at[idx_v.at[pl.ds(t * _TILE + h * _H, _H)]],
          buf.at[s, pl.ds(h * _H, _H)], sems_in[s]).wait()

  def start_out(t, s):
    row0 = wid * rows_per_w + t * _ROWS_T
    pltpu.async_copy(bufo.at[s], out_hbm.at[pl.ds(row0, _ROWS_T)],
                     sems_out[s])

  def wait_out(t, s):
    row0 = wid * rows_per_w + t * _ROWS_T
    pltpu.make_async_copy(bufo.at[s], out_hbm.at[pl.ds(row0, _ROWS_T)],
                          sems_out[s]).wait()

  def compute_tile(t, s):
    base_r = lax.rem(t * _TILE, _P)

    @plsc.parallel_loop(0, _TILE, 1, unroll=4)
    def tok(j):
      r0 = base_r + j
      r = jnp.where(r0 >= _P, r0 - _P, r0)
      x = []
      for dd in range(_D // 16):
        sl = pl.ds(dd * 16, 16)
        x.append(buf[s, j, sl] + avpos[r, sl])
      sm = (x[0] + x[1]) + (x[2] + x[3])
      sq = (x[0] * x[0] + x[1] * x[1]) + (x[2] * x[2] + x[3] * x[3])
      ssum = jnp.sum(sm)
      qsum = jnp.sum(sq)
      mean = ssum * (1.0 / _D)
      var = qsum * (1.0 / _D) - mean * mean + _EPS
      iv = lax.bitcast_convert_type(var, jnp.int32)
      iv = jnp.int32(0x5F3759DF) - lax.shift_right_logical(iv, 1)
      y = lax.bitcast_convert_type(iv, jnp.float32)
      h = var * 0.5
      y = y * (1.5 - h * y * y)
      y = y * (1.5 - h * y * y)
      y = y * (1.5 - h * y * y)
      c0 = mean * y
      q = j // _L
      l = j - q * _L
      for dd in range(_D // 16):
        bufo[s, q, l, pl.ds(dd * 16, 16)] = (
            (x[dd] * y - c0) * g[dd] + b[dd])

  start_gather(0, 0)

  def outer(tt, _):
    for s in range(2):
      t = tt * 2 + s
      wait_gather(t, s)
      compute_tile(t, s)
      start_out(t, s)
      nxt = 1 - s
      if s == 0:
        @pl.when(tt >= 1)
        def _():
          wait_out(t - 1, nxt)
        start_gather(t + 1, nxt)
      else:
        @pl.when(tt < tiles_per_w // 2 - 1)
        def _():
          wait_out(t - 1, nxt)
          start_gather(t + 1, nxt)
    return 0

  lax.fori_loop(0, tiles_per_w // 2, outer, 0)
  wait_out(tiles_per_w - 2, 0)
  wait_out(tiles_per_w - 1, 1)


@jax.jit
def kernel(X_scan, token_table, av_table, pos_table, ln_gamma, ln_beta):
  rows, seq = X_scan.shape
  n = rows * seq
  xr = X_scan.reshape(n).astype(jnp.int32)

  mesh = plsc.VectorSubcoreMesh(
      core_axis_name="c", subcore_axis_name="s",
      num_cores=_NC, num_subcores=_NS)

  run = pl.kernel(
      _kernel_body,
      out_type=jax.ShapeDtypeStruct((rows, seq, _D), jnp.float32),
      mesh=mesh,
      scratch_types=[
          pltpu.VMEM((n // _NW,), jnp.int32),
          pltpu.VMEM((_A, _D), jnp.float32),
          pltpu.VMEM((_L, _D), jnp.float32),
          pltpu.VMEM((_P, _D), jnp.float32),
          pltpu.VMEM((_D,), jnp.float32),
          pltpu.VMEM((_D,), jnp.float32),
          pltpu.VMEM((2, _TILE, _D), jnp.float32),
          pltpu.VMEM((2, _ROWS_T, _L, _D), jnp.float32),
          pltpu.SemaphoreType.DMA,
          pltpu.SemaphoreType.DMA,
          pltpu.SemaphoreType.DMA,
          pltpu.SemaphoreType.DMA,
      ],
      compiler_params=pltpu.CompilerParams(
          needs_layout_passes=False, use_tc_tiling_on_sc=False),
  )
  return run(xr, token_table, av_table, pos_table, ln_gamma, ln_beta)


# speedup vs baseline: 1.0603x; 1.0005x over previous
import functools

import jax
import jax.numpy as jnp
from jax import lax
from jax.experimental import pallas as pl
from jax.experimental.layout import Layout, with_layout_constraint
from jax.experimental.pallas import tpu as pltpu
from jax.experimental.pallas import tpu_sc as plsc

_A = 26
_L = 20
_D = 64
_P = _A * _L
_TILE = 160
_ROWS_T = _TILE // _L
_NC = 2
_NS = 16
_NW = _NC * _NS
_EPS = 1e-5


def _kernel_body(xr_hbm, tok_hbm, av_hbm, pos_hbm, gam_hbm, bet_hbm, out_hbm,
                 idx_v, av_v, pos_v, avpos, gam_v, bet_v, buf, bufo,
                 sem_in0, sem_in1, sem_out0, sem_out1):
  n_per_w = xr_hbm.shape[0] // _NW
  tiles_per_w = n_per_w // _TILE
  rows_per_w = n_per_w // _L
  wid = lax.axis_index("s") * _NC + lax.axis_index("c")

  pltpu.sync_copy(xr_hbm.at[pl.ds(wid * n_per_w, n_per_w)], idx_v)
  pltpu.sync_copy(av_hbm, av_v)
  pltpu.sync_copy(pos_hbm, pos_v)
  pltpu.sync_copy(gam_hbm, gam_v)
  pltpu.sync_copy(bet_hbm, bet_v)

  def build_avpos(r, _):
    a = r // _L
    l = r - a * _L
    for dd in range(_D // 16):
      sl = pl.ds(dd * 16, 16)
      avpos[r, sl] = av_v[a, sl] + pos_v[l, sl]
    return 0
  lax.fori_loop(0, _P, build_avpos, 0)

  g = [gam_v[pl.ds(dd * 16, 16)] for dd in range(_D // 16)]
  b = [bet_v[pl.ds(dd * 16, 16)] for dd in range(_D // 16)]

  sems_in = (sem_in0, sem_in1)
  sems_out = (sem_out0, sem_out1)
  _H = _TILE // 2

  def start_gather(t, s):
    for h in range(2):
      pltpu.async_copy(
          tok_hbm.at[idx_v.at[pl.ds(t * _TILE + h * _H, _H)]],
          buf.at[s, pl.ds(h * _H, _H)], sems_in[s])

  def wait_gather(t, s):
    for h in range(2):
      pltpu.make_async_copy(
          tok_hbm.at[idx_v.at[pl.ds(t * _TILE + h * _H, _H)]],
          buf.at[s, pl.ds(h * _H, _H)], sems_in[s]).wait()

  def start_out(t, s):
    row0 = wid * rows_per_w + t * _ROWS_T
    pltpu.async_copy(bufo.at[s], out_hbm.at[pl.ds(row0, _ROWS_T)],
                     sems_out[s])

  def wait_out(t, s):
    row0 = wid * rows_per_w + t * _ROWS_T
    pltpu.make_async_copy(bufo.at[s], out_hbm.at[pl.ds(row0, _ROWS_T)],
                          sems_out[s]).wait()

  def compute_tile(t, s):
    base_r = lax.rem(t * _TILE, _P)

    @plsc.parallel_loop(0, _TILE, 1, unroll=4)
    def tok(j):
      r0 = base_r + j
      r = jnp.where(r0 >= _P, r0 - _P, r0)
      x = []
      for dd in range(_D // 16):
        sl = pl.ds(dd * 16, 16)
        x.append(buf[s, j, sl] + avpos[r, sl])
      sm = (x[0] + x[1]) + (x[2] + x[3])
      sq = (x[0] * x[0] + x[1] * x[1]) + (x[2] * x[2] + x[3] * x[3])
      ssum = jnp.sum(sm)
      qsum = jnp.sum(sq)
      mean = ssum * (1.0 / _D)
      var = qsum * (1.0 / _D) - mean * mean + _EPS
      iv = lax.bitcast_convert_type(var, jnp.int32)
      iv = jnp.int32(0x5F3759DF) - lax.shift_right_logical(iv, 1)
      y = lax.bitcast_convert_type(iv, jnp.float32)
      h = var * 0.5
      y = y * (1.5 - h * y * y)
      y = y * (1.5 - h * y * y)
      y = y * (1.5 - h * y * y)
      c0 = mean * y
      q = j // _L
      l = j - q * _L
      for dd in range(_D // 16):
        bufo[s, q, l, pl.ds(dd * 16, 16)] = (
            (x[dd] * y - c0) * g[dd] + b[dd])

  start_gather(0, 0)

  def outer(tt, _):
    for s in range(2):
      t = tt * 2 + s
      wait_gather(t, s)
      compute_tile(t, s)
      start_out(t, s)
      nxt = 1 - s
      if s == 0:
        @pl.when(tt >= 1)
        def _():
          wait_out(t - 1, nxt)
        start_gather(t + 1, nxt)
      else:
        @pl.when(tt < tiles_per_w // 2 - 1)
        def _():
          wait_out(t - 1, nxt)
          start_gather(t + 1, nxt)
    return 0

  lax.fori_loop(0, tiles_per_w // 2, outer, 0)
  wait_out(tiles_per_w - 2, 0)
  wait_out(tiles_per_w - 1, 1)


@jax.jit
def kernel(X_scan, token_table, av_table, pos_table, ln_gamma, ln_beta):
  rows, seq = X_scan.shape
  n = rows * seq
  xr = X_scan.reshape(n).astype(jnp.int32)

  mesh = plsc.VectorSubcoreMesh(
      core_axis_name="c", subcore_axis_name="s",
      num_cores=_NC, num_subcores=_NS)

  run = pl.kernel(
      _kernel_body,
      out_type=jax.ShapeDtypeStruct((rows, seq, _D), jnp.float32),
      mesh=mesh,
      scratch_types=[
          pltpu.VMEM((n // _NW,), jnp.int32),
          pltpu.VMEM((_A, _D), jnp.float32),
          pltpu.VMEM((_L, _D), jnp.float32),
          pltpu.VMEM((_P, _D), jnp.float32),
          pltpu.VMEM((_D,), jnp.float32),
          pltpu.VMEM((_D,), jnp.float32),
          pltpu.VMEM((2, _TILE, _D), jnp.float32),
          pltpu.VMEM((2, _ROWS_T, _L, _D), jnp.float32),
          pltpu.SemaphoreType.DMA,
          pltpu.SemaphoreType.DMA,
          pltpu.SemaphoreType.DMA,
          pltpu.SemaphoreType.DMA,
      ],
      compiler_params=pltpu.CompilerParams(
          needs_layout_passes=False, use_tc_tiling_on_sc=False),
  )
  out = run(xr, token_table, av_table, pos_table, ln_gamma, ln_beta)
  return with_layout_constraint(
      out, Layout(major_to_minor=(0, 1, 2), tiling=()))
